# baseline (speedup 1.0000x reference)
import functools

import jax
import jax.numpy as jnp
from jax import lax
from jax.experimental import pallas as pl
from jax.experimental.pallas import tpu as pltpu
from jax.experimental.pallas import tpu_sc as plsc

B = 8
N = 32768
K = 2048
NCHUNK = N // K


def _cmpex(key, idx, lane, j, dir_desc):
    bit = (lane & j) != 0
    pk = jnp.where(bit, jnp.roll(key, j, axis=-1), jnp.roll(key, -j, axis=-1))
    pi = jnp.where(bit, jnp.roll(idx, j, axis=-1), jnp.roll(idx, -j, axis=-1))
    self_wins = (key > pk) | ((key == pk) & (idx < pi))
    want_big = dir_desc ^ bit
    take_self = ~(self_wins ^ want_big)
    return jnp.where(take_self, key, pk), jnp.where(take_self, idx, pi)


def _topk_tc_body(scores_ref, out_scores_ref, out_idx_ref):
    s = scores_ref[:]
    b = lax.bitcast_convert_type(s, jnp.int32)
    key = jnp.where(b < 0, b ^ jnp.int32(0x7FFFFFFF), b)
    idx = lax.broadcasted_iota(jnp.int32, (B, N), 1)
    key = key.reshape(B, NCHUNK, K)
    idx = idx.reshape(B, NCHUNK, K)
    lane = lax.broadcasted_iota(jnp.int32, (B, NCHUNK, K), 2)
    chunk_odd = (lax.broadcasted_iota(jnp.int32, (B, NCHUNK, K), 1) & 1) == 1

    k = 2
    while k <= K:
        j = k // 2
        while j >= 1:
            dir_desc = ((lane & k) == 0) ^ chunk_odd
            key, idx = _cmpex(key, idx, lane, j, dir_desc)
            j //= 2
        k *= 2

    m = NCHUNK
    while m > 1:
        m //= 2
        kk = key.reshape(B, m, 2, K)
        ii = idx.reshape(B, m, 2, K)
        a_k, a_i = kk[:, :, 0], ii[:, :, 0]
        b_k, b_i = kk[:, :, 1], ii[:, :, 1]
        a_wins = (a_k > b_k) | ((a_k == b_k) & (a_i < b_i))
        key = jnp.where(a_wins, a_k, b_k)
        idx = jnp.where(a_wins, a_i, b_i)
        lane = lane[:, :m]
        chunk_odd = (lax.broadcasted_iota(jnp.int32, (B, m, K), 1) & 1) == 1
        j = K // 2
        while j >= 1:
            key, idx = _cmpex(key, idx, lane, j, ~chunk_odd)
            j //= 2

    key = key.reshape(B, K)
    idx = idx.reshape(B, K)
    sb = jnp.where(key < 0, key ^ jnp.int32(0x7FFFFFFF), key)
    out_scores_ref[:] = lax.bitcast_convert_type(sb, jnp.float32)
    out_idx_ref[:] = idx


_topk_tc = pl.pallas_call(
    _topk_tc_body,
    out_shape=(
        jax.ShapeDtypeStruct((B, K), jnp.float32),
        jax.ShapeDtypeStruct((B, K), jnp.int32),
    ),
)


_SC_CORES = 2
_SC_SUBCORES = 16


@functools.lru_cache(maxsize=None)
def _make_sc_gather():
    nw = _SC_CORES * _SC_SUBCORES
    per_w = (2 * B * K) // nw
    n_sub = per_w // 128
    mesh = plsc.VectorSubcoreMesh(core_axis_name="c", subcore_axis_name="s")

    @functools.partial(
        pl.kernel,
        mesh=mesh,
        out_type=jax.ShapeDtypeStruct((2 * B * K,), jnp.float32),
        scratch_types=[
            pltpu.VMEM((n_sub, 128), jnp.int32),
            pltpu.VMEM((per_w,), jnp.float32),
            pltpu.SemaphoreType.DMA,
        ],
    )
    def gather_k(kpts_hbm, idx_hbm, out_hbm, idx_v, vals_v, sem):
        wid = lax.axis_index("s") * _SC_CORES + lax.axis_index("c")
        base = wid * per_w
        pltpu.sync_copy(idx_hbm.at[wid], idx_v)
        copies = [
            pltpu.async_copy(
                kpts_hbm.at[idx_v.at[j]], vals_v.at[pl.ds(j * 128, 128)], sem
            )
            for j in range(n_sub)
        ]
        for c in copies:
            c.wait()
        pltpu.sync_copy(vals_v, out_hbm.at[pl.ds(base, per_w)])

    def gather(kpts, flat_idx):
        idx2 = jnp.stack((flat_idx * 2, flat_idx * 2 + 1), axis=-1)
        return gather_k(kpts.reshape(2 * B * N), idx2.reshape(nw, n_sub, 128))

    return gather


def kernel(keypoints, scores):
    top_scores, top_idx = _topk_tc(scores)
    flat_idx = top_idx + jnp.arange(B, dtype=jnp.int32)[:, None] * N
    top_kpts = _make_sc_gather()(keypoints, flat_idx).reshape(B, K, 2)
    return (top_kpts, top_scores)


# speedup vs baseline: 1.5152x; 1.5152x over previous
import functools

import jax
import jax.numpy as jnp
from jax import lax
from jax.experimental import pallas as pl
from jax.experimental.pallas import tpu as pltpu
from jax.experimental.pallas import tpu_sc as plsc

B = 8
N = 32768
K = 2048

_SC_CORES = 2
_SC_SUBCORES = 16

_I32_MIN = jnp.iinfo(jnp.int32).min
_I32_MAX = jnp.iinfo(jnp.int32).max


def _thresh_tc_body(scores_ref, ctrl_ref):
    s = scores_ref[:]
    b = lax.bitcast_convert_type(s, jnp.int32)
    key = jnp.where(b < 0, b ^ jnp.int32(0x7FFFFFFF), b)

    lo0 = jnp.full((B, 1), _I32_MIN, jnp.int32)
    hi0 = jnp.full((B, 1), _I32_MAX, jnp.int32)

    def step(_, carry):
        lo, hi = carry
        t = hi - lo
        half = ((t >> 1) & jnp.int32(0x7FFFFFFF)) + (t & 1)
        mid = lo + half
        cnt = jnp.sum((key >= mid).astype(jnp.int32), axis=1, keepdims=True)
        ge = cnt >= K
        return jnp.where(ge, mid, lo), jnp.where(ge, hi, mid - 1)

    lo, _ = lax.fori_loop(0, 32, step, (lo0, hi0))
    t_row = lo
    g_row = jnp.sum((key > t_row).astype(jnp.int32), axis=1, keepdims=True)
    lane = lax.broadcasted_iota(jnp.int32, (B, 128), 1)
    ctrl_ref[:] = jnp.where(
        lane == 0,
        jnp.broadcast_to(t_row, (B, 128)),
        jnp.broadcast_to(g_row, (B, 128)),
    )


_thresh_tc = pl.pallas_call(
    _thresh_tc_body,
    out_shape=jax.ShapeDtypeStruct((B, 128), jnp.int32),
)


@functools.lru_cache(maxsize=None)
def _make_sc_compact():
    mesh = plsc.VectorSubcoreMesh(core_axis_name="c", subcore_axis_name="s")

    @functools.partial(
        pl.kernel,
        mesh=mesh,
        compiler_params=pltpu.CompilerParams(needs_layout_passes=False),
        out_type=(
            jax.ShapeDtypeStruct((B, K), jnp.int32),
            jax.ShapeDtypeStruct((B, K), jnp.int32),
        ),
        scratch_types=[
            pltpu.VMEM((N,), jnp.float32),
            pltpu.VMEM((128,), jnp.int32),
            pltpu.VMEM((K + 16,), jnp.int32),
            pltpu.VMEM((K + 16,), jnp.int32),
            pltpu.SemaphoreType.DMA,
        ],
    )
    def compact_k(scores_hbm, ctrl_hbm, okey_hbm, oidx_hbm, sv, cv, okv, oiv, sem):
        wid = lax.axis_index("s") * _SC_CORES + lax.axis_index("c")

        @pl.when(wid < B)
        def _():
            pltpu.sync_copy(scores_hbm.at[wid], sv)
            pltpu.sync_copy(ctrl_hbm.at[wid], cv)
            head = cv[pl.ds(0, 16)]
            t_vec = jnp.broadcast_to(head[0], (16,))
            g_cnt = head[1]

            def body(i, carry):
                gt_off, eq_off = carry
                v = sv[pl.ds(i * 16, 16)]
                bb = lax.bitcast_convert_type(v, jnp.int32)
                keyv = jnp.where(bb < 0, bb ^ jnp.int32(0x7FFFFFFF), bb)
                iv = lax.broadcasted_iota(jnp.int32, (16,), 0) + i * 16
                gt = keyv > t_vec
                eq = keyv == t_vec
                ngt = jnp.sum(gt.astype(jnp.int32), axis=0)
                plsc.store_compressed(okv.at[pl.ds(gt_off, 16)], keyv, mask=gt)
                plsc.store_compressed(oiv.at[pl.ds(gt_off, 16)], iv, mask=gt)
                c = plsc.cumsum(eq.astype(jnp.int32))
                keep = eq & ((eq_off + c) <= K)
                nk = jnp.sum(keep.astype(jnp.int32), axis=0)
                plsc.store_compressed(okv.at[pl.ds(eq_off, 16)], keyv, mask=keep)
                plsc.store_compressed(oiv.at[pl.ds(eq_off, 16)], iv, mask=keep)
                return gt_off + ngt, eq_off + nk

            lax.fori_loop(0, N // 16, body, (jnp.int32(0), g_cnt))
            pltpu.sync_copy(okv.at[pl.ds(0, K)], okey_hbm.at[wid])
            pltpu.sync_copy(oiv.at[pl.ds(0, K)], oidx_hbm.at[wid])

    return compact_k


def _cmpex(key, idx, lane, j, dir_desc):
    bit = (lane & j) != 0
    pk = jnp.where(bit, jnp.roll(key, j, axis=-1), jnp.roll(key, -j, axis=-1))
    pi = jnp.where(bit, jnp.roll(idx, j, axis=-1), jnp.roll(idx, -j, axis=-1))
    self_wins = (key > pk) | ((key == pk) & (idx < pi))
    want_big = dir_desc ^ bit
    take_self = ~(self_wins ^ want_big)
    return jnp.where(take_self, key, pk), jnp.where(take_self, idx, pi)


def _sort_tc_body(key_ref, idx_ref, out_scores_ref, out_idx_ref):
    key = key_ref[:]
    idx = idx_ref[:]
    lane = lax.broadcasted_iota(jnp.int32, (B, K), 1)

    k = 2
    while k <= K:
        j = k // 2
        while j >= 1:
            dir_desc = (lane & k) == 0 if k < K else True
            key, idx = _cmpex(key, idx, lane, j, dir_desc)
            j //= 2
        k *= 2

    sb = jnp.where(key < 0, key ^ jnp.int32(0x7FFFFFFF), key)
    out_scores_ref[:] = lax.bitcast_convert_type(sb, jnp.float32)
    out_idx_ref[:] = idx


_sort_tc = pl.pallas_call(
    _sort_tc_body,
    out_shape=(
        jax.ShapeDtypeStruct((B, K), jnp.float32),
        jax.ShapeDtypeStruct((B, K), jnp.int32),
    ),
)


@functools.lru_cache(maxsize=None)
def _make_sc_gather():
    nw = _SC_CORES * _SC_SUBCORES
    per_w = (2 * B * K) // nw
    n_sub = per_w // 128
    mesh = plsc.VectorSubcoreMesh(core_axis_name="c", subcore_axis_name="s")

    @functools.partial(
        pl.kernel,
        mesh=mesh,
        out_type=jax.ShapeDtypeStruct((2 * B * K,), jnp.float32),
        scratch_types=[
            pltpu.VMEM((n_sub, 128), jnp.int32),
            pltpu.VMEM((per_w,), jnp.float32),
            pltpu.SemaphoreType.DMA,
        ],
    )
    def gather_k(kpts_hbm, idx_hbm, out_hbm, idx_v, vals_v, sem):
        wid = lax.axis_index("s") * _SC_CORES + lax.axis_index("c")
        base = wid * per_w
        pltpu.sync_copy(idx_hbm.at[wid], idx_v)
        copies = [
            pltpu.async_copy(
                kpts_hbm.at[idx_v.at[j]], vals_v.at[pl.ds(j * 128, 128)], sem
            )
            for j in range(n_sub)
        ]
        for c in copies:
            c.wait()
        pltpu.sync_copy(vals_v, out_hbm.at[pl.ds(base, per_w)])

    def gather(kpts, flat_idx):
        idx2 = jnp.stack((flat_idx * 2, flat_idx * 2 + 1), axis=-1)
        return gather_k(kpts.reshape(2 * B * N), idx2.reshape(nw, n_sub, 128))

    return gather


def kernel(keypoints, scores):
    ctrl = _thresh_tc(scores)
    ckey, cidx = _make_sc_compact()(scores, ctrl)
    top_scores, top_idx = _sort_tc(ckey, cidx)
    flat_idx = top_idx + jnp.arange(B, dtype=jnp.int32)[:, None] * N
    top_kpts = _make_sc_gather()(keypoints, flat_idx).reshape(B, K, 2)
    return (top_kpts, top_scores)
